# Initial kernel scaffold; baseline (speedup 1.0000x reference)
import jax
import jax.numpy as jnp
from jax.experimental import pallas as pl


def kernel(cls_emb, rel_emb, nf1, nf2, nf3, nf4, dis, top, nf3_neg, nf_inclusion, nf_chain, radius, dataset):
    raise NotImplementedError("write your pallas kernel here")


# speedup vs baseline: 2.3633x; 2.3633x over previous
import jax
import jax.numpy as jnp
from jax import lax
from jax.experimental import pallas as pl
from jax.experimental.pallas import tpu as pltpu
from jax.experimental.pallas import tpu_sc as plsc

EMB = 128
MARGIN = 0.1
INF = 5.0
B = 4096
L = 16
NW = 32
BPT = B // NW
NBLK = BPT // L
NC1 = 5
NR1 = 1
NC2 = 6
NR2 = 3
NT3 = 2


def _sqrt16(s):
    i = plsc.bitcast(s, jnp.int32)
    y = plsc.bitcast(jnp.int32(0x5F3759DF) - lax.shift_right_arithmetic(i, 1),
                     jnp.float32)
    for _ in range(3):
        y = y * (1.5 - ((0.5 * s) * y) * y)
    return s * y


def _relu(x):
    return jnp.maximum(x, 0.0)


def _sc_body(xs_hbm, ts_hbm, rel_hbm, nf1_h, nf2_h, nf3_h, nf4_h, top_h,
             nn_h, rad_h, out_h,
             nf1_v, nf2_v, nf3_v, nf4_v, nn_v, top_v, rad_v,
             ic1_v, ir1_v, ic2_v, ir2_v, it3_v,
             c1_v, r1_v, c2_v, r2_v, tv1_v, tv2_v, tv3_v, tot_v, sem):
    wid = lax.axis_index("s") * 2 + lax.axis_index("c")
    base = wid * BPT
    iota = lax.iota(jnp.int32, L)

    pltpu.sync_copy(nf1_h.at[pl.ds(base, BPT), :], nf1_v)
    pltpu.sync_copy(nf2_h.at[pl.ds(base, BPT), :], nf2_v)
    pltpu.sync_copy(nf3_h.at[pl.ds(base, BPT), :], nf3_v)
    pltpu.sync_copy(nf4_h.at[pl.ds(base, BPT), :], nf4_v)
    pltpu.sync_copy(nn_h.at[pl.ds(base, BPT), :], nn_v)
    pltpu.sync_copy(top_h.at[pl.ds(base, BPT)], top_v)
    pltpu.sync_copy(rad_h.at[pl.ds(base, BPT)], rad_v)

    total = jnp.zeros((L,), jnp.float32)

    for blk in range(NBLK):
        rows = iota + blk * L

        def col(ref, c):
            return plsc.load_gather(ref, [rows, jnp.full((L,), c, jnp.int32)])

        ic1_v[pl.ds(0 * L, L)] = col(nf1_v, 0)
        ic1_v[pl.ds(1 * L, L)] = col(nf1_v, 2)
        ic1_v[pl.ds(2 * L, L)] = col(nf2_v, 0)
        ic1_v[pl.ds(3 * L, L)] = col(nf2_v, 1)
        ic1_v[pl.ds(4 * L, L)] = col(nf2_v, 2)
        ir1_v[pl.ds(0 * L, L)] = col(nf1_v, 1)
        ic2_v[pl.ds(0 * L, L)] = col(nf3_v, 0)
        ic2_v[pl.ds(1 * L, L)] = col(nf3_v, 2)
        ic2_v[pl.ds(2 * L, L)] = col(nf4_v, 1)
        ic2_v[pl.ds(3 * L, L)] = col(nf4_v, 2)
        ic2_v[pl.ds(4 * L, L)] = col(nn_v, 0)
        ic2_v[pl.ds(5 * L, L)] = col(nn_v, 2)
        ir2_v[pl.ds(0 * L, L)] = col(nf3_v, 1)
        ir2_v[pl.ds(1 * L, L)] = col(nf4_v, 0)
        ir2_v[pl.ds(2 * L, L)] = col(nn_v, 1)
        it3_v[pl.ds(0 * L, L)] = top_v[pl.ds(blk * L, L)]
        it3_v[pl.ds(1 * L, L)] = rad_v[pl.ds(blk * L, L)]

        d1 = pltpu.async_copy(xs_hbm.at[ic1_v], c1_v, sem)
        d2 = pltpu.async_copy(rel_hbm.at[ir1_v], r1_v, sem)
        d3 = pltpu.async_copy(xs_hbm.at[ic2_v], c2_v, sem)
        d4 = pltpu.async_copy(rel_hbm.at[ir2_v], r2_v, sem)
        d5 = pltpu.async_copy(ts_hbm.at[ic1_v], tv1_v, sem)
        d6 = pltpu.async_copy(ts_hbm.at[ic2_v], tv2_v, sem)
        d7 = pltpu.async_copy(ts_hbm.at[it3_v], tv3_v, sem)
        for d in (d1, d2, d3, d4, d5, d6, d7):
            d.wait()

        zero = jnp.zeros((L,), jnp.float32)

        def eb1(e, accs):
            (aA, aB, aC, aD, aE, a1, aCD, aCE, aDE) = accs
            ce = jnp.full((L,), e, jnp.int32)

            def lg(ref, s):
                return plsc.load_gather(ref, [iota + s * L, ce])

            vA = lg(c1_v, 0); vB = lg(c1_v, 1); vC = lg(c1_v, 2)
            vD = lg(c1_v, 3); vE = lg(c1_v, 4); w1 = lg(r1_v, 0)
            aA = aA + vA * vA
            aB = aB + vB * vB
            aC = aC + vC * vC
            aD = aD + vD * vD
            aE = aE + vE * vE
            t = vA + w1 - vB
            a1 = a1 + t * t
            t = vD - vC
            aCD = aCD + t * t
            t = vE - vC
            aCE = aCE + t * t
            t = vE - vD
            aDE = aDE + t * t
            return (aA, aB, aC, aD, aE, a1, aCD, aCE, aDE)

        (aA, aB, aC, aD, aE, a1, aCD, aCE, aDE) = lax.fori_loop(
            0, EMB, eb1, (zero,) * 9)

        def eb2(e, accs):
            (aF, aG, aH, aI, aK, aL, a3, a4, a5) = accs
            ce = jnp.full((L,), e, jnp.int32)

            def lg(ref, s):
                return plsc.load_gather(ref, [iota + s * L, ce])

            vF = lg(c2_v, 0); vG = lg(c2_v, 1); vH = lg(c2_v, 2)
            vI = lg(c2_v, 3); vK = lg(c2_v, 4); vL = lg(c2_v, 5)
            w3 = lg(r2_v, 0); w4 = lg(r2_v, 1); w5 = lg(r2_v, 2)
            aF = aF + vF * vF
            aG = aG + vG * vG
            aH = aH + vH * vH
            aI = aI + vI * vI
            aK = aK + vK * vK
            aL = aL + vL * vL
            t = vF + w3 - vG
            a3 = a3 + t * t
            t = vH - w4 - vI
            a4 = a4 + t * t
            t = vK + w5 - vL
            a5 = a5 + t * t
            return (aF, aG, aH, aI, aK, aL, a3, a4, a5)

        (aF, aG, aH, aI, aK, aL, a3, a4, a5) = lax.fori_loop(
            0, EMB, eb2, (zero,) * 9)

        rA = _relu(tv1_v[pl.ds(0 * L, L)])
        rB = _relu(tv1_v[pl.ds(1 * L, L)])
        rC = _relu(tv1_v[pl.ds(2 * L, L)])
        rD = _relu(tv1_v[pl.ds(3 * L, L)])
        rE = _relu(tv1_v[pl.ds(4 * L, L)])
        rF = _relu(tv2_v[pl.ds(0 * L, L)])
        rG = _relu(tv2_v[pl.ds(1 * L, L)])
        rH = _relu(tv2_v[pl.ds(2 * L, L)])
        rI = _relu(tv2_v[pl.ds(3 * L, L)])
        rK = _relu(tv2_v[pl.ds(4 * L, L)])
        rL = _relu(tv2_v[pl.ds(5 * L, L)])
        rJ = _relu(tv3_v[pl.ds(0 * L, L)])
        tP = tv3_v[pl.ds(1 * L, L)]

        def reg(a):
            return jnp.abs(_sqrt16(a) - 1.0)

        loss = (
            _relu(_sqrt16(a1) + rA - rB - MARGIN) + reg(aA) + reg(aB)
            + _relu(_sqrt16(aCD) - (rC + rD) - MARGIN)
            + _relu(_sqrt16(aCE) - rC - MARGIN)
            + _relu(_sqrt16(aDE) - rD - MARGIN)
            + _relu(jnp.minimum(rC, rD) - rE - MARGIN)
            + reg(aC) + reg(aD) + reg(aE)
            + _relu(_sqrt16(a3) + rF - rG - MARGIN) + reg(aF) + reg(aG)
            + _relu(_sqrt16(a4) - (rH + rI) - MARGIN) + reg(aH) + reg(aI)
            + jnp.abs(rJ - INF)
            + (MARGIN - (_sqrt16(a5) - rK - rL)) + reg(aK) + reg(aL)
            - jnp.minimum(tP, 0.0)
        )
        total = total + loss

    tot_v[...] = total
    pltpu.sync_copy(tot_v, out_h.at[wid])


def kernel(cls_emb, rel_emb, nf1, nf2, nf3, nf4, dis, top, nf3_neg,
           nf_inclusion, nf_chain, radius, dataset):
    xs = cls_emb[:, :EMB]
    ts = cls_emb[:, EMB]
    mesh = plsc.VectorSubcoreMesh(core_axis_name="c", subcore_axis_name="s")
    cp = pltpu.CompilerParams(needs_layout_passes=False,
                              use_tc_tiling_on_sc=False)
    sc = pl.kernel(
        _sc_body,
        out_type=jax.ShapeDtypeStruct((NW, L), jnp.float32),
        mesh=mesh,
        compiler_params=cp,
        scratch_types=[
            pltpu.VMEM((BPT, 3), jnp.int32),
            pltpu.VMEM((BPT, 3), jnp.int32),
            pltpu.VMEM((BPT, 3), jnp.int32),
            pltpu.VMEM((BPT, 3), jnp.int32),
            pltpu.VMEM((BPT, 3), jnp.int32),
            pltpu.VMEM((BPT,), jnp.int32),
            pltpu.VMEM((BPT,), jnp.int32),
            pltpu.VMEM((NC1 * L,), jnp.int32),
            pltpu.VMEM((NR1 * L,), jnp.int32),
            pltpu.VMEM((NC2 * L,), jnp.int32),
            pltpu.VMEM((NR2 * L,), jnp.int32),
            pltpu.VMEM((NT3 * L,), jnp.int32),
            pltpu.VMEM((NC1 * L, EMB), jnp.float32),
            pltpu.VMEM((NR1 * L, EMB), jnp.float32),
            pltpu.VMEM((NC2 * L, EMB), jnp.float32),
            pltpu.VMEM((NR2 * L, EMB), jnp.float32),
            pltpu.VMEM((NC1 * L,), jnp.float32),
            pltpu.VMEM((NC2 * L,), jnp.float32),
            pltpu.VMEM((NT3 * L,), jnp.float32),
            pltpu.VMEM((L,), jnp.float32),
            pltpu.SemaphoreType.DMA,
        ],
    )
    part = sc(xs, ts, rel_emb,
              nf1.astype(jnp.int32), nf2.astype(jnp.int32),
              nf3.astype(jnp.int32), nf4.astype(jnp.int32),
              top.astype(jnp.int32), nf3_neg.astype(jnp.int32),
              radius.astype(jnp.int32))
    return (jnp.sum(part) / jnp.float32(B)) ** 2
